# Initial kernel scaffold; baseline (speedup 1.0000x reference)
import jax
import jax.numpy as jnp
from jax.experimental import pallas as pl


def kernel(x, adj, W, a):
    raise NotImplementedError("write your pallas kernel here")


# speedup vs baseline: 5255.0317x; 5255.0317x over previous
import functools

import jax
import jax.numpy as jnp
from jax.experimental import pallas as pl
from jax.experimental.pallas import tpu as pltpu

N = 2048
FIN = 128
FOUT = 32
BLK = 128
GRID = N // BLK


def _body(x_ref, adj_ref, a_ref, w_ref, out_ref, h_s, t_s):
    i = pl.program_id(0)

    @pl.when(i == 0)
    def _init():
        h = jnp.dot(x_ref[...], w_ref[...], preferred_element_type=jnp.float32)
        h_s[...] = h
        a1 = a_ref[0, FOUT:]
        t_s[...] = jnp.dot(h, a1, preferred_element_type=jnp.float32)[None, :]

    h = h_s[...]
    h_blk = h_s[pl.ds(i * BLK, BLK), :]
    a0 = a_ref[0, :FOUT]
    s_blk = jnp.dot(h_blk, a0, preferred_element_type=jnp.float32)
    u = s_blk[:, None] + t_s[0, :][None, :]
    lr = jnp.maximum(u, 0.2 * u)
    e = jnp.exp(-jnp.clip(lr, -50.0, 50.0))
    w = jnp.where(adj_ref[...] != 0, e, 0.0)
    numer = jnp.dot(w, h, preferred_element_type=jnp.float32)
    denom = jnp.sum(w, axis=1)
    hp = numer / denom[:, None]
    out_ref[...] = jnp.where(hp > 0, hp, jnp.exp(hp) - 1.0)


@jax.jit
def kernel(x, adj, W, a):
    return pl.pallas_call(
        _body,
        grid=(GRID,),
        in_specs=[
            pl.BlockSpec((N, FIN), lambda i: (0, 0)),
            pl.BlockSpec((BLK, N), lambda i: (i, 0)),
            pl.BlockSpec((1, 2 * FOUT), lambda i: (0, 0)),
            pl.BlockSpec((FIN, FOUT), lambda i: (0, 0)),
        ],
        out_specs=pl.BlockSpec((BLK, FOUT), lambda i: (i, 0)),
        out_shape=jax.ShapeDtypeStruct((N, FOUT), jnp.float32),
        scratch_shapes=[
            pltpu.VMEM((N, FOUT), jnp.float32),
            pltpu.VMEM((1, N), jnp.float32),
        ],
        compiler_params=pltpu.CompilerParams(
            dimension_semantics=("arbitrary",),
        ),
    )(x, adj, a, W)
